# baseline (speedup 1.0000x reference)
import functools

import jax
import jax.numpy as jnp
from jax import lax
from jax.experimental import pallas as pl
from jax.experimental.pallas import tpu as pltpu
from jax.experimental.pallas import tpu_sc as plsc

D = 64
SCALE = 8.0


@functools.lru_cache(maxsize=None)
def _make_transpose(V: int):
    info = plsc.get_sparse_core_info()
    NC, NS = info.num_cores, info.num_subcores
    NW = NC * NS
    W = 128
    n_starts = V // W
    tail = V - n_starts * W
    starts_per_w = (n_starts + NW - 1) // NW
    mesh = plsc.VectorSubcoreMesh(core_axis_name="c", subcore_axis_name="s")

    @functools.partial(
        pl.kernel,
        mesh=mesh,
        out_type=jax.ShapeDtypeStruct((V, 128), jnp.float32),
        scratch_types=[
            pltpu.VMEM((D, W), jnp.float32),
            pltpu.VMEM((W, 128), jnp.float32),
            pltpu.VMEM((tail if tail else 8, D), jnp.float32),
            pltpu.SemaphoreType.DMA,
        ],
        compiler_params=pltpu.CompilerParams(use_tc_tiling_on_sc=True, needs_layout_passes=False),
    )
    def k(tt_hbm, tail_hbm, out_hbm, slab_v, tslab_v, tail_v, sem):
        wid = lax.axis_index("s") * NC + lax.axis_index("c")

        def block_body(i, _):
            blk = wid * starts_per_w + i

            @pl.when(blk < n_starts)
            def _():
                r0 = pl.multiple_of(blk * W, W)
                pltpu.sync_copy(tt_hbm.at[:, pl.ds(r0, W)], slab_v)

                def trans_row(r, _):
                    for cg in range(D // 16):
                        col_ids = lax.broadcasted_iota(jnp.int32, (16,), 0) + (
                            cg * 16
                        )
                        row_ids = jnp.full((16,), r, jnp.int32)
                        vals = plsc.load_gather(slab_v, [col_ids, row_ids])
                        tslab_v[r, pl.ds(cg * 16, 16)] = vals
                    return 0

                lax.fori_loop(0, W, trans_row, 0)
                pltpu.sync_copy(tslab_v, out_hbm.at[pl.ds(r0, W)])

            return 0

        lax.fori_loop(0, starts_per_w, block_body, 0)

        if tail:
            @pl.when(wid == NW - 1)
            def _():
                pltpu.sync_copy(tail_hbm, tail_v)

                def tail_row(r, _):
                    for cg in range(D // 16):
                        sl = pl.ds(cg * 16, 16)
                        tslab_v[r, sl] = tail_v[r, sl]
                    return 0

                lax.fori_loop(0, tail, tail_row, 0)
                pltpu.sync_copy(
                    tslab_v.at[pl.ds(0, tail)],
                    out_hbm.at[pl.ds(n_starts * W, tail)],
                )

    return k


@functools.lru_cache(maxsize=None)
def _make_gather(NB: int, S: int, V: int, C: int):
    info = plsc.get_sparse_core_info()
    NC, NS = info.num_cores, info.num_subcores
    NW = NC * NS
    B = NB * S
    assert B % NW == 0
    b_per_w = B // NW
    assert b_per_w % C == 0 and C % S == 0
    n_chunks = b_per_w // C
    R = C // S
    mesh = plsc.VectorSubcoreMesh(core_axis_name="c", subcore_axis_name="s")

    @functools.partial(
        pl.kernel,
        mesh=mesh,
        out_type=jax.ShapeDtypeStruct((NB, S, D), jnp.float32),
        scratch_types=[
            pltpu.VMEM((C,), jnp.int32),
            pltpu.VMEM((C, 128), jnp.float32),
            pltpu.VMEM((C, D), jnp.float32),
            pltpu.SemaphoreType.DMA,
        ],
        compiler_params=pltpu.CompilerParams(use_tc_tiling_on_sc=True, needs_layout_passes=False),
    )
    def k(idx_hbm, table_hbm, out_hbm, idx_v, rows_v, outc_v, sem):
        wid = lax.axis_index("s") * NC + lax.axis_index("c")
        base = wid * b_per_w

        def chunk_body(cidx, _):
            off = base + cidx * C
            b0 = off // S
            pltpu.sync_copy(idx_hbm.at[pl.ds(off, C)], idx_v)
            pltpu.async_copy(table_hbm.at[idx_v], rows_v, sem).wait()

            def scale_row(s, _):
                for j in range(D // 16):
                    sl = pl.ds(j * 16, 16)
                    outc_v[s, sl] = rows_v[s, sl] * SCALE
                return 0

            lax.fori_loop(0, C, scale_row, 0)
            for a in range(R):
                pltpu.sync_copy(
                    outc_v.at[pl.ds(a * S, S)], out_hbm.at[b0 + a]
                )
            return 0

        lax.fori_loop(0, n_chunks, chunk_body, 0)

    return k


def kernel(x, table):
    NB, S = x.shape
    V = table.shape[0]
    flat = x.reshape(NB * S).astype(jnp.int32)
    tail_start = (V // 128) * 128
    table_p = _make_transpose(V)(table.T, table[tail_start:])
    out = _make_gather(NB, S, V, S)(flat, table_p)
    return out


# speedup vs baseline: 2.5690x; 2.5690x over previous
import functools

import jax
import jax.numpy as jnp
from jax import lax
from jax.experimental import pallas as pl
from jax.experimental.pallas import tpu as pltpu
from jax.experimental.pallas import tpu_sc as plsc

D = 64
SCALE = 8.0


@functools.lru_cache(maxsize=None)
def _make_kernel(NB: int, S: int, R: int):
    info = plsc.get_sparse_core_info()
    NC, NS = info.num_cores, info.num_subcores
    NW = NC * NS
    assert NB % NW == 0
    rows_per_w = NB // NW
    assert rows_per_w % R == 0
    n_chunks = rows_per_w // R
    C = R * S
    assert C % 16 == 0
    mesh = plsc.VectorSubcoreMesh(core_axis_name="c", subcore_axis_name="s")

    @functools.partial(
        pl.kernel,
        mesh=mesh,
        out_type=jax.ShapeDtypeStruct((NB, S, D), jnp.float32),
        scratch_types=[
            pltpu.VMEM((C,), jnp.int32),
            pltpu.VMEM((C, D), jnp.float32),
            pltpu.SemaphoreType.DMA,
        ],
        compiler_params=pltpu.CompilerParams(use_tc_tiling_on_sc=True),
    )
    def k(idx_hbm, table_hbm, out_hbm, idx_v, rows_v, gsem):
        wid = lax.axis_index("s") * NC + lax.axis_index("c")
        base_row = wid * rows_per_w

        def chunk_body(cidx, _):
            b0 = base_row + cidx * R
            off = b0 * S
            pltpu.sync_copy(idx_hbm.at[pl.ds(off, C)], idx_v)

            def fire(blk, _):
                k0 = blk * 16
                v = idx_v[pl.ds(k0, 16)]
                for i in range(16):
                    pltpu.async_copy(
                        table_hbm.at[v[i]], rows_v.at[k0 + i], gsem
                    )
                return 0

            lax.fori_loop(0, C // 16, fire, 0)

            for a in range(R):
                pltpu.make_async_copy(
                    out_hbm.at[b0 + a], rows_v.at[pl.ds(a * S, S)], gsem
                ).wait()

            def scale_row(s, _):
                for j in range(D // 16):
                    sl = pl.ds(j * 16, 16)
                    rows_v[s, sl] = rows_v[s, sl] * SCALE
                return 0

            lax.fori_loop(0, C, scale_row, 0)
            for a in range(R):
                pltpu.sync_copy(
                    rows_v.at[pl.ds(a * S, S)], out_hbm.at[b0 + a]
                )
            return 0

        lax.fori_loop(0, n_chunks, chunk_body, 0)

    return k


def kernel(x, table):
    NB, S = x.shape
    flat = x.reshape(NB * S).astype(jnp.int32)
    out = _make_kernel(NB, S, 4)(flat, table)
    return out
